# Initial kernel scaffold; baseline (speedup 1.0000x reference)
import jax
import jax.numpy as jnp
from jax.experimental import pallas as pl


def kernel(x, table):
    raise NotImplementedError("write your pallas kernel here")


# speedup vs baseline: 9.5858x; 9.5858x over previous
import functools

import jax
import jax.numpy as jnp
from jax import lax
from jax.experimental import pallas as pl
from jax.experimental.pallas import tpu as pltpu
from jax.experimental.pallas import tpu_sc as plsc

VOCAB = 1000
D = 128
W = 10
P = 128
NC, NS = 2, 16
NW = NC * NS


def _scale_body(table_ref, out_ref):
    w = pl.program_id(0)
    scale = (10.0 - w.astype(jnp.float32)) / 55.0
    out_ref[...] = (table_ref[...] * scale)[None]


def _build_scaled(table):
    return pl.pallas_call(
        _scale_body,
        grid=(W,),
        in_specs=[pl.BlockSpec((VOCAB, D), lambda w: (0, 0))],
        out_specs=pl.BlockSpec((1, VOCAB, D), lambda w: (w, 0, 0)),
        out_shape=jax.ShapeDtypeStruct((W, VOCAB, D), jnp.float32),
    )(table)


def _make_sc_call(n_pos):
    n_chunks = n_pos // P
    chunks_per_worker = n_chunks // NW
    mesh = plsc.VectorSubcoreMesh(
        core_axis_name="c", subcore_axis_name="s", num_cores=NC, num_subcores=NS
    )

    @functools.partial(
        pl.kernel,
        mesh=mesh,
        out_type=jax.ShapeDtypeStruct((n_pos, D), jnp.float32),
        scratch_types=[
            pltpu.VMEM((W, P), jnp.int32),
            pltpu.VMEM((P, D), jnp.float32),
            pltpu.SemaphoreType.DMA,
        ],
    )
    def sc_kernel(idx_hbm, scaled_hbm, out_hbm, idx_v, acc_v, sem):
        wid = lax.axis_index("s") * NC + lax.axis_index("c")

        def body(i, carry):
            c = wid * chunks_per_worker + i
            pltpu.sync_copy(idx_hbm.at[c], idx_v)
            pltpu.async_copy(scaled_hbm.at[idx_v.at[0]], acc_v, sem).wait()
            for w in range(1, W):
                pltpu.async_copy(
                    scaled_hbm.at[idx_v.at[w]], acc_v, sem, add=True
                ).wait()
            pltpu.sync_copy(acc_v, out_hbm.at[pl.ds(c * P, P)])
            return carry

        lax.fori_loop(0, chunks_per_worker, body, 0)

    return sc_kernel


def kernel(x, table):
    B, T, _ = x.shape
    n_pos = B * T
    scaled = _build_scaled(table).reshape(W * VOCAB, D)
    xf = x.reshape(n_pos // P, P, W).astype(jnp.int32)
    offs = jnp.arange(W, dtype=jnp.int32) * VOCAB
    idx = jnp.transpose(xf, (0, 2, 1)) + offs[None, :, None]
    out = _make_sc_call(n_pos)(idx, scaled)
    return out.reshape(B, T, D)


# speedup vs baseline: 13.0289x; 1.3592x over previous
import functools

import jax
import jax.numpy as jnp
from jax import lax
from jax.experimental import pallas as pl
from jax.experimental.pallas import tpu as pltpu
from jax.experimental.pallas import tpu_sc as plsc

VOCAB = 1000
D = 128
W = 10
P = 128
NC, NS = 2, 16
NW = NC * NS


def _scale_body(table_ref, out_ref):
    w = pl.program_id(0)
    scale = (10.0 - w.astype(jnp.float32)) / 55.0
    out_ref[...] = (table_ref[...] * scale)[None]


def _build_scaled(table):
    return pl.pallas_call(
        _scale_body,
        grid=(W,),
        in_specs=[pl.BlockSpec((VOCAB, D), lambda w: (0, 0))],
        out_specs=pl.BlockSpec((1, VOCAB, D), lambda w: (w, 0, 0)),
        out_shape=jax.ShapeDtypeStruct((W, VOCAB, D), jnp.float32),
    )(table)


def _make_sc_call(n_pos):
    n_chunks = n_pos // P
    chunks_per_worker = n_chunks // NW
    mesh = plsc.VectorSubcoreMesh(
        core_axis_name="c", subcore_axis_name="s", num_cores=NC, num_subcores=NS
    )

    @functools.partial(
        pl.kernel,
        mesh=mesh,
        out_type=jax.ShapeDtypeStruct((n_pos, D), jnp.float32),
        scratch_types=[
            pltpu.VMEM((W, P), jnp.int32),
            pltpu.VMEM((P, D), jnp.float32),
            pltpu.SemaphoreType.DMA,
        ],
    )
    def sc_kernel(idx_hbm, scaled_hbm, out_hbm, idx_v, acc_v, sem):
        wid = lax.axis_index("s") * NC + lax.axis_index("c")

        def body(i, carry):
            c = wid * chunks_per_worker + i
            pltpu.sync_copy(idx_hbm.at[c], idx_v)
            pltpu.async_copy(scaled_hbm.at[idx_v.at[0]], acc_v, sem).wait()
            copies = [
                pltpu.async_copy(scaled_hbm.at[idx_v.at[w]], acc_v, sem, add=True)
                for w in range(1, W)
            ]
            for cp in copies:
                cp.wait()
            pltpu.sync_copy(acc_v, out_hbm.at[pl.ds(c * P, P)])
            return carry

        lax.fori_loop(0, chunks_per_worker, body, 0)

    return sc_kernel


def kernel(x, table):
    B, T, _ = x.shape
    n_pos = B * T
    scaled = _build_scaled(table).reshape(W * VOCAB, D)
    xf = x.reshape(n_pos // P, P, W).astype(jnp.int32)
    offs = jnp.arange(W, dtype=jnp.int32) * VOCAB
    idx = jnp.transpose(xf, (0, 2, 1)) + offs[None, :, None]
    out = _make_sc_call(n_pos)(idx, scaled)
    return out.reshape(B, T, D)


# speedup vs baseline: 14.4745x; 1.1110x over previous
import functools

import jax
import jax.numpy as jnp
from jax import lax
from jax.experimental import pallas as pl
from jax.experimental.pallas import tpu as pltpu
from jax.experimental.pallas import tpu_sc as plsc

VOCAB = 1000
D = 128
W = 10
P = 128
NC, NS = 2, 16
NW = NC * NS


def _scale_body(table_ref, out_ref):
    w = pl.program_id(0)
    scale = (10.0 - w.astype(jnp.float32)) / 55.0
    out_ref[...] = (table_ref[...] * scale)[None]


def _build_scaled(table):
    return pl.pallas_call(
        _scale_body,
        grid=(W,),
        in_specs=[pl.BlockSpec((VOCAB, D), lambda w: (0, 0))],
        out_specs=pl.BlockSpec((1, VOCAB, D), lambda w: (w, 0, 0)),
        out_shape=jax.ShapeDtypeStruct((W, VOCAB, D), jnp.float32),
    )(table)


NB = 4


def _make_sc_call(n_pos):
    n_chunks = n_pos // P
    cpw = n_chunks // NW
    mesh = plsc.VectorSubcoreMesh(
        core_axis_name="c", subcore_axis_name="s", num_cores=NC, num_subcores=NS
    )

    @functools.partial(
        pl.kernel,
        mesh=mesh,
        out_type=jax.ShapeDtypeStruct((n_pos, D), jnp.float32),
        scratch_types=(
            [pltpu.VMEM((W, P), jnp.int32) for _ in range(NB)]
            + [pltpu.VMEM((P, D), jnp.float32) for _ in range(NB)]
            + [pltpu.SemaphoreType.DMA for _ in range(3 * NB)]
        ),
    )
    def sc_kernel(idx_hbm, scaled_hbm, out_hbm, *scr):
        idxs = scr[0:NB]
        accs = scr[NB : 2 * NB]
        gsems = scr[2 * NB : 3 * NB]
        isems = scr[3 * NB : 4 * NB]
        osems = scr[4 * NB : 5 * NB]
        wid = lax.axis_index("s") * NC + lax.axis_index("c")
        base = wid * cpw

        def fire_idx(c, k):
            pltpu.async_copy(idx_hbm.at[c], idxs[k], isems[k])

        def wait_idx(k):
            pltpu.make_async_copy(idx_hbm.at[base], idxs[k], isems[k]).wait()

        def fire_ow(k):
            pltpu.async_copy(scaled_hbm.at[idxs[k].at[0]], accs[k], gsems[k])

        def fire_adds(k):
            for w in range(1, W):
                pltpu.async_copy(
                    scaled_hbm.at[idxs[k].at[w]], accs[k], gsems[k], add=True
                )

        def wait_g(k, n):
            for _ in range(n):
                pltpu.make_async_copy(
                    scaled_hbm.at[pl.ds(0, P)], accs[k], gsems[k]
                ).wait()

        def fire_out(c, k):
            pltpu.async_copy(accs[k], out_hbm.at[pl.ds(c * P, P)], osems[k])

        def wait_out(k):
            pltpu.make_async_copy(
                out_hbm.at[pl.ds(0, P)], accs[k], osems[k]
            ).wait()

        def body(c, k):
            km1 = (k - 1) % NB
            km2 = (k - 2) % NB
            wait_idx(k)
            wait_out(k)
            fire_ow(k)
            wait_g(km1, 1)
            fire_adds(km1)
            wait_g(km2, W - 1)
            fire_out(c - 2, km2)
            fire_idx(c + 2, km2)

        fire_idx(base + 0, 0)
        fire_idx(base + 1, 1)
        wait_idx(0)
        fire_ow(0)
        fire_idx(base + 2, 2)
        wait_idx(1)
        fire_ow(1)
        wait_g(0, 1)
        fire_adds(0)
        fire_idx(base + 3, 3)
        for k in (2, 3):
            wait_idx(k)
            fire_ow(k)
            wait_g(k - 1, 1)
            fire_adds(k - 1)
            wait_g(k - 2, W - 1)
            fire_out(base + k - 2, k - 2)
            fire_idx(base + k + 2, k - 2)

        def loop_body(j, carry):
            c0 = base + NB + NB * j
            for k in range(NB):
                body(c0 + k, k)
            return carry

        lax.fori_loop(0, (cpw - 2 * NB + 2) // NB, loop_body, 0)

        for k in (0, 1):
            c = base + cpw - 2 + k
            wait_idx(k)
            wait_out(k)
            fire_ow(k)
            wait_g((k - 1) % NB, 1)
            fire_adds((k - 1) % NB)
            wait_g((k - 2) % NB, W - 1)
            fire_out(c - 2, (k - 2) % NB)
        wait_g(1, 1)
        fire_adds(1)
        wait_g(0, W - 1)
        fire_out(base + cpw - 2, 0)
        wait_g(1, W - 1)
        fire_out(base + cpw - 1, 1)
        for k in (2, 3, 0, 1):
            wait_out(k)

    return sc_kernel


def kernel(x, table):
    B, T, _ = x.shape
    n_pos = B * T
    scaled = _build_scaled(table).reshape(W * VOCAB, D)
    xf = x.reshape(n_pos // P, P, W).astype(jnp.int32)
    offs = jnp.arange(W, dtype=jnp.int32) * VOCAB
    idx = jnp.transpose(xf, (0, 2, 1)) + offs[None, :, None]
    out = _make_sc_call(n_pos)(idx, scaled)
    return out.reshape(B, T, D)
